# baseline (speedup 1.0000x reference)
import functools

import jax
import jax.numpy as jnp
from jax import lax
from jax.experimental import pallas as pl
from jax.experimental.pallas import tpu as pltpu
from jax.experimental.pallas import tpu_sc as plsc

_NW = 32
_L = 16
_CHUNK = 128
_NSEM = 7


def _make_kernel(V, F):
    fpt = -(-F // _NW)
    fpt = -(-fpt // _CHUNK) * _CHUNK
    nchunks = fpt // _CHUNK
    assert nchunks % _NSEM == 0
    nsteps = fpt // _L
    last = F - (_NW - 1) * fpt

    mesh = plsc.VectorSubcoreMesh(core_axis_name="c", subcore_axis_name="s")

    out_t = jax.ShapeDtypeStruct((F,), jnp.float32)

    @functools.partial(
        pl.kernel,
        out_type=(out_t, out_t, out_t),
        mesh=mesh,
        scratch_types=(
            [pltpu.VMEM((3 * nchunks, _CHUNK), jnp.int32)]
            + [pltpu.VMEM((fpt,), jnp.float32) for _ in range(12)]
            + [pltpu.SemaphoreType.DMA for _ in range(_NSEM)]
        ),
    )
    def face_normals(vx_hbm, vy_hbm, vz_hbm, fidx_hbm,
                     ox_hbm, oy_hbm, oz_hbm,
                     idx_v,
                     gax, gay, gaz, gbx, gby, gbz, gcx, gcy, gcz,
                     ovx, ovy, ovz, *sems):
        g_v = ((gax, gay, gaz), (gbx, gby, gbz), (gcx, gcy, gcz))
        o_v = (ovx, ovy, ovz)
        wid = lax.axis_index("s") * 2 + lax.axis_index("c")
        base = wid * fpt

        for v in range(3):
            pltpu.sync_copy(fidx_hbm.at[v, wid],
                            idx_v.at[pl.ds(v * nchunks, nchunks)])

        coord_hbm = (vx_hbm, vy_hbm, vz_hbm)

        def fire(j, sem):
            sl = pl.ds(j * _CHUNK, _CHUNK)
            for v in range(3):
                idx = idx_v.at[v * nchunks + j]
                for k in range(3):
                    pltpu.async_copy(coord_hbm[k].at[idx],
                                     g_v[v][k].at[sl], sem)

        def drain(j, sem):
            sl = pl.ds(j * _CHUNK, _CHUNK)
            for v in range(3):
                for k in range(3):
                    pltpu.make_async_copy(coord_hbm[k].at[pl.ds(0, _CHUNK)],
                                          g_v[v][k].at[sl], sem).wait()

        def step(i, carry):
            sl = pl.ds(i * _L, _L)
            ax = gax[sl]
            ay = gay[sl]
            az = gaz[sl]
            bx = gbx[sl]
            by = gby[sl]
            bz = gbz[sl]
            cx = gcx[sl]
            cy = gcy[sl]
            cz = gcz[sl]
            e1x = bx - ax
            e1y = by - ay
            e1z = bz - az
            e2x = cx - ax
            e2y = cy - ay
            e2z = cz - az
            nx = e1y * e2z - e1z * e2y
            ny = e1z * e2x - e1x * e2z
            nz = e1x * e2y - e1y * e2x
            nn = nx * nx + ny * ny + nz * nz
            bits = lax.bitcast_convert_type(nn, jnp.int32)
            y = lax.bitcast_convert_type(
                jnp.int32(0x5F3759DF) - (bits >> 1), jnp.float32)
            y = y * (1.5 - 0.5 * nn * y * y)
            y = y * (1.5 - 0.5 * nn * y * y)
            y = y * (1.5 - 0.5 * nn * y * y)
            norm = jnp.maximum(nn * y, jnp.float32(1e-12))
            ovx[sl] = nx / norm
            ovy[sl] = ny / norm
            ovz[sl] = nz / norm
            return carry

        ngroups = nchunks // _NSEM
        spc = _CHUNK // _L
        for p in range(_NSEM):
            fire(p, sems[p])

        def group(g, carry):
            for p in range(_NSEM):
                j = g * _NSEM + p
                drain(j, sems[p])

                @pl.when(g < ngroups - 1)
                def _():
                    fire(j + _NSEM, sems[p])

                lax.fori_loop(j * spc, j * spc + spc, step, 0)
            return carry

        lax.fori_loop(0, ngroups, group, 0)

        out_hbm = (ox_hbm, oy_hbm, oz_hbm)

        @pl.when(wid < _NW - 1)
        def _():
            for k in range(3):
                pltpu.sync_copy(o_v[k],
                                out_hbm[k].at[pl.ds(base, fpt)])

        @pl.when(wid == _NW - 1)
        def _():
            for k in range(3):
                pltpu.sync_copy(o_v[k].at[pl.ds(0, last)],
                                out_hbm[k].at[pl.ds(base, last)])

    return face_normals, fpt, nchunks


def kernel(verts, faces):
    V = verts.shape[0]
    F = faces.shape[0]
    fn, fpt, nchunks = _make_kernel(V, F)
    verts = verts.astype(jnp.float32)
    vx = verts[:, 0]
    vy = verts[:, 1]
    vz = verts[:, 2]
    pad = _NW * fpt - F
    ft = faces.astype(jnp.int32).T
    ft = jnp.pad(ft, ((0, 0), (0, pad)))
    fidx = ft.reshape(3, _NW, fpt // _CHUNK, _CHUNK)
    ox, oy, oz = fn(vx, vy, vz, fidx)
    return jnp.stack([ox, oy, oz], axis=1)


# speedup vs baseline: 1.0011x; 1.0011x over previous
import functools

import jax
import jax.numpy as jnp
from jax import lax
from jax.experimental import pallas as pl
from jax.experimental.pallas import tpu as pltpu
from jax.experimental.pallas import tpu_sc as plsc

_NW = 32
_L = 16
_CHUNK = 128
_NSEM = 7


def _make_kernel(V, F):
    fpt = -(-F // _NW)
    fpt = -(-fpt // _CHUNK) * _CHUNK
    nchunks = fpt // _CHUNK
    assert nchunks % _NSEM == 0
    nsteps = fpt // _L
    last = F - (_NW - 1) * fpt

    mesh = plsc.VectorSubcoreMesh(core_axis_name="c", subcore_axis_name="s")

    out_t = jax.ShapeDtypeStruct((F,), jnp.float32)

    @functools.partial(
        pl.kernel,
        out_type=(out_t, out_t, out_t),
        mesh=mesh,
        scratch_types=(
            [pltpu.VMEM((3 * nchunks, _CHUNK), jnp.int32)]
            + [pltpu.VMEM((fpt,), jnp.float32) for _ in range(12)]
            + [pltpu.SemaphoreType.DMA for _ in range(_NSEM)]
        ),
    )
    def face_normals(vx_hbm, vy_hbm, vz_hbm, fidx_hbm,
                     ox_hbm, oy_hbm, oz_hbm,
                     idx_v,
                     gax, gay, gaz, gbx, gby, gbz, gcx, gcy, gcz,
                     ovx, ovy, ovz, *sems):
        g_v = ((gax, gay, gaz), (gbx, gby, gbz), (gcx, gcy, gcz))
        o_v = (ovx, ovy, ovz)
        wid = lax.axis_index("s") * 2 + lax.axis_index("c")
        base = wid * fpt

        for v in range(3):
            pltpu.sync_copy(fidx_hbm.at[v, wid],
                            idx_v.at[pl.ds(v * nchunks, nchunks)])

        coord_hbm = (vx_hbm, vy_hbm, vz_hbm)

        def fire(j, sem):
            sl = pl.ds(j * _CHUNK, _CHUNK)
            for v in range(3):
                idx = idx_v.at[v * nchunks + j]
                for k in range(3):
                    pltpu.async_copy(coord_hbm[k].at[idx],
                                     g_v[v][k].at[sl], sem)

        def drain(j, sem):
            pltpu.make_async_copy(vx_hbm.at[pl.ds(0, 9 * _CHUNK)],
                                  gax.at[pl.ds(0, 9 * _CHUNK)], sem).wait()

        def step(i, carry):
            sl = pl.ds(i * _L, _L)
            ax = gax[sl]
            ay = gay[sl]
            az = gaz[sl]
            bx = gbx[sl]
            by = gby[sl]
            bz = gbz[sl]
            cx = gcx[sl]
            cy = gcy[sl]
            cz = gcz[sl]
            e1x = bx - ax
            e1y = by - ay
            e1z = bz - az
            e2x = cx - ax
            e2y = cy - ay
            e2z = cz - az
            nx = e1y * e2z - e1z * e2y
            ny = e1z * e2x - e1x * e2z
            nz = e1x * e2y - e1y * e2x
            nn = nx * nx + ny * ny + nz * nz
            bits = lax.bitcast_convert_type(nn, jnp.int32)
            y = lax.bitcast_convert_type(
                jnp.int32(0x5F3759DF) - (bits >> 1), jnp.float32)
            y = y * (1.5 - 0.5 * nn * y * y)
            y = y * (1.5 - 0.5 * nn * y * y)
            y = y * (1.5 - 0.5 * nn * y * y)
            norm = jnp.maximum(nn * y, jnp.float32(1e-12))
            ovx[sl] = nx / norm
            ovy[sl] = ny / norm
            ovz[sl] = nz / norm
            return carry

        ngroups = nchunks // _NSEM
        spc = _CHUNK // _L
        for p in range(_NSEM):
            fire(p, sems[p])

        def group(g, carry):
            for p in range(_NSEM):
                j = g * _NSEM + p
                drain(j, sems[p])

                @pl.when(g < ngroups - 1)
                def _():
                    fire(j + _NSEM, sems[p])

                lax.fori_loop(j * spc, j * spc + spc, step, 0)
            return carry

        lax.fori_loop(0, ngroups, group, 0)

        out_hbm = (ox_hbm, oy_hbm, oz_hbm)

        @pl.when(wid < _NW - 1)
        def _():
            for k in range(3):
                pltpu.sync_copy(o_v[k],
                                out_hbm[k].at[pl.ds(base, fpt)])

        @pl.when(wid == _NW - 1)
        def _():
            for k in range(3):
                pltpu.sync_copy(o_v[k].at[pl.ds(0, last)],
                                out_hbm[k].at[pl.ds(base, last)])

    return face_normals, fpt, nchunks


def kernel(verts, faces):
    V = verts.shape[0]
    F = faces.shape[0]
    fn, fpt, nchunks = _make_kernel(V, F)
    verts = verts.astype(jnp.float32)
    vx = verts[:, 0]
    vy = verts[:, 1]
    vz = verts[:, 2]
    pad = _NW * fpt - F
    ft = faces.astype(jnp.int32).T
    ft = jnp.pad(ft, ((0, 0), (0, pad)))
    fidx = ft.reshape(3, _NW, fpt // _CHUNK, _CHUNK)
    ox, oy, oz = fn(vx, vy, vz, fidx)
    return jnp.stack([ox, oy, oz], axis=1)


# speedup vs baseline: 1.0214x; 1.0203x over previous
import functools

import jax
import jax.numpy as jnp
from jax import lax
from jax.experimental import pallas as pl
from jax.experimental.pallas import tpu as pltpu
from jax.experimental.pallas import tpu_sc as plsc

_NW = 32
_L = 16
_CHUNK = 128
_NG = 7


def _make_kernel(V, F):
    fpt = -(-F // _NW)
    fpt = -(-fpt // _CHUNK) * _CHUNK
    nchunks = fpt // _CHUNK
    nsteps = fpt // _L
    last = F - (_NW - 1) * fpt

    mesh = plsc.VectorSubcoreMesh(core_axis_name="c", subcore_axis_name="s")

    out_t = jax.ShapeDtypeStruct((F,), jnp.float32)

    @functools.partial(
        pl.kernel,
        out_type=(out_t, out_t, out_t),
        mesh=mesh,
        scratch_types=(
            [pltpu.VMEM((3 * nchunks, _CHUNK), jnp.int32)]
            + [pltpu.VMEM((fpt,), jnp.float32) for _ in range(12)]
            + [pltpu.SemaphoreType.DMA for _ in range(_NG)]
        ),
    )
    def face_normals(vx_hbm, vy_hbm, vz_hbm, fidx_hbm,
                     ox_hbm, oy_hbm, oz_hbm,
                     idx_v,
                     gax, gay, gaz, gbx, gby, gbz, gcx, gcy, gcz,
                     ovx, ovy, ovz, *sems):
        g_v = ((gax, gay, gaz), (gbx, gby, gbz), (gcx, gcy, gcz))
        o_v = (ovx, ovy, ovz)
        wid = lax.axis_index("s") * 2 + lax.axis_index("c")
        base = wid * fpt

        pltpu.sync_copy(fidx_hbm.at[wid], idx_v)

        coord_hbm = (vx_hbm, vy_hbm, vz_hbm)

        cpg = nchunks // _NG

        for g in range(_NG):
            def fire(j, carry, sem=sems[g]):
                sl = pl.ds(j * _CHUNK, _CHUNK)
                for v in range(3):
                    idx = idx_v.at[v * nchunks + j]
                    for k in range(3):
                        pltpu.async_copy(coord_hbm[k].at[idx],
                                         g_v[v][k].at[sl], sem)
                return carry

            lax.fori_loop(g * cpg, (g + 1) * cpg, fire, 0)

        def step(i, carry):
            sl = pl.ds(i * _L, _L)
            ax = gax[sl]
            ay = gay[sl]
            az = gaz[sl]
            bx = gbx[sl]
            by = gby[sl]
            bz = gbz[sl]
            cx = gcx[sl]
            cy = gcy[sl]
            cz = gcz[sl]
            e1x = bx - ax
            e1y = by - ay
            e1z = bz - az
            e2x = cx - ax
            e2y = cy - ay
            e2z = cz - az
            nx = e1y * e2z - e1z * e2y
            ny = e1z * e2x - e1x * e2z
            nz = e1x * e2y - e1y * e2x
            nn = nx * nx + ny * ny + nz * nz
            bits = lax.bitcast_convert_type(nn, jnp.int32)
            y = lax.bitcast_convert_type(
                jnp.int32(0x5F3759DF) - (bits >> 1), jnp.float32)
            y = y * (1.5 - 0.5 * nn * y * y)
            y = y * (1.5 - 0.5 * nn * y * y)
            y = y * (1.5 - 0.5 * nn * y * y)
            norm = jnp.maximum(nn * y, jnp.float32(1e-12))
            ovx[sl] = nx / norm
            ovy[sl] = ny / norm
            ovz[sl] = nz / norm
            return carry

        spc = _CHUNK // _L
        for g in range(_NG):
            for _ in range(cpg):
                pltpu.make_async_copy(vx_hbm.at[pl.ds(0, 9 * _CHUNK)],
                                      gax.at[pl.ds(0, 9 * _CHUNK)],
                                      sems[g]).wait()
            lax.fori_loop(g * cpg * spc, (g + 1) * cpg * spc, step, 0)

        out_hbm = (ox_hbm, oy_hbm, oz_hbm)

        @pl.when(wid < _NW - 1)
        def _():
            for k in range(3):
                pltpu.sync_copy(o_v[k],
                                out_hbm[k].at[pl.ds(base, fpt)])

        @pl.when(wid == _NW - 1)
        def _():
            for k in range(3):
                pltpu.sync_copy(o_v[k].at[pl.ds(0, last)],
                                out_hbm[k].at[pl.ds(base, last)])

    return face_normals, fpt, nchunks


def kernel(verts, faces):
    V = verts.shape[0]
    F = faces.shape[0]
    fn, fpt, nchunks = _make_kernel(V, F)
    verts = verts.astype(jnp.float32)
    vx = verts[:, 0]
    vy = verts[:, 1]
    vz = verts[:, 2]
    pad = _NW * fpt - F
    fcols = jnp.concatenate(
        [faces.astype(jnp.int32), jnp.zeros((pad, 3), jnp.int32)])
    fidx = fcols.reshape(_NW, fpt, 3).transpose(0, 2, 1)
    fidx = fidx.reshape(_NW, 3 * nchunks, _CHUNK)
    ox, oy, oz = fn(vx, vy, vz, fidx)
    return jnp.stack([ox, oy, oz], axis=1)


# speedup vs baseline: 1.0480x; 1.0260x over previous
import functools

import jax
import jax.numpy as jnp
from jax import lax
from jax.experimental import pallas as pl
from jax.experimental.pallas import tpu as pltpu
from jax.experimental.pallas import tpu_sc as plsc

_NW = 32
_L = 16
_CHUNK = 128


def _make_kernel(V, F):
    fpt = -(-F // _NW)
    fpt = -(-fpt // _CHUNK) * _CHUNK
    nchunks = fpt // _CHUNK
    nsteps = fpt // _L
    last = F - (_NW - 1) * fpt

    mesh = plsc.VectorSubcoreMesh(core_axis_name="c", subcore_axis_name="s")

    out_t = jax.ShapeDtypeStruct((F,), jnp.float32)

    @functools.partial(
        pl.kernel,
        out_type=(out_t, out_t, out_t),
        mesh=mesh,
        scratch_types=(
            [pltpu.VMEM((3 * nchunks, _CHUNK), jnp.int32)]
            + [pltpu.VMEM((fpt,), jnp.float32) for _ in range(12)]
            + [pltpu.SemaphoreType.DMA]
        ),
    )
    def face_normals(vx_hbm, vy_hbm, vz_hbm, fidx_hbm,
                     ox_hbm, oy_hbm, oz_hbm,
                     idx_v,
                     gax, gay, gaz, gbx, gby, gbz, gcx, gcy, gcz,
                     ovx, ovy, ovz, sem):
        g_v = ((gax, gay, gaz), (gbx, gby, gbz), (gcx, gcy, gcz))
        o_v = (ovx, ovy, ovz)
        wid = lax.axis_index("s") * 2 + lax.axis_index("c")
        base = wid * fpt

        pltpu.sync_copy(fidx_hbm.at[wid], idx_v)

        coord_hbm = (vx_hbm, vy_hbm, vz_hbm)

        def fire(j, carry):
            sl = pl.ds(j * _CHUNK, _CHUNK)
            for v in range(3):
                idx = idx_v.at[v * nchunks + j]
                for k in range(3):
                    pltpu.async_copy(coord_hbm[k].at[idx],
                                     g_v[v][k].at[sl], sem)
            return carry

        lax.fori_loop(0, nchunks, fire, 0)
        for v in range(3):
            for k in range(3):
                pltpu.make_async_copy(coord_hbm[k].at[pl.ds(0, fpt)],
                                      g_v[v][k], sem).wait()

        def step(i, carry):
            sl = pl.ds(i * _L, _L)
            ax = gax[sl]
            ay = gay[sl]
            az = gaz[sl]
            bx = gbx[sl]
            by = gby[sl]
            bz = gbz[sl]
            cx = gcx[sl]
            cy = gcy[sl]
            cz = gcz[sl]
            e1x = bx - ax
            e1y = by - ay
            e1z = bz - az
            e2x = cx - ax
            e2y = cy - ay
            e2z = cz - az
            nx = e1y * e2z - e1z * e2y
            ny = e1z * e2x - e1x * e2z
            nz = e1x * e2y - e1y * e2x
            nn = nx * nx + ny * ny + nz * nz
            bits = lax.bitcast_convert_type(nn, jnp.int32)
            y = lax.bitcast_convert_type(
                jnp.int32(0x5F3759DF) - (bits >> 1), jnp.float32)
            y = y * (1.5 - 0.5 * nn * y * y)
            y = y * (1.5 - 0.5 * nn * y * y)
            y = y * (1.5 - 0.5 * nn * y * y)
            norm = jnp.maximum(nn * y, jnp.float32(1e-12))
            ovx[sl] = nx / norm
            ovy[sl] = ny / norm
            ovz[sl] = nz / norm
            return carry

        lax.fori_loop(0, nsteps, step, 0)

        out_hbm = (ox_hbm, oy_hbm, oz_hbm)

        @pl.when(wid < _NW - 1)
        def _():
            for k in range(3):
                pltpu.sync_copy(o_v[k],
                                out_hbm[k].at[pl.ds(base, fpt)])

        @pl.when(wid == _NW - 1)
        def _():
            for k in range(3):
                pltpu.sync_copy(o_v[k].at[pl.ds(0, last)],
                                out_hbm[k].at[pl.ds(base, last)])

    return face_normals, fpt, nchunks


def kernel(verts, faces):
    V = verts.shape[0]
    F = faces.shape[0]
    fn, fpt, nchunks = _make_kernel(V, F)
    verts = verts.astype(jnp.float32)
    vx = verts[:, 0]
    vy = verts[:, 1]
    vz = verts[:, 2]
    pad = _NW * fpt - F
    fcols = jnp.concatenate(
        [faces.astype(jnp.int32), jnp.zeros((pad, 3), jnp.int32)])
    fidx = fcols.reshape(_NW, fpt, 3).transpose(0, 2, 1)
    fidx = fidx.reshape(_NW, 3 * nchunks, _CHUNK)
    ox, oy, oz = fn(vx, vy, vz, fidx)
    return jnp.stack([ox, oy, oz], axis=1)
